# Initial kernel scaffold; baseline (speedup 1.0000x reference)
import jax
import jax.numpy as jnp
from jax.experimental import pallas as pl


def kernel(x, edge_index, batch, W1, b1, W2, b2):
    raise NotImplementedError("write your pallas kernel here")


# speedup vs baseline: 9.5401x; 9.5401x over previous
import jax
import jax.numpy as jnp
from jax import lax
from jax.experimental import pallas as pl
from jax.experimental.pallas import tpu as pltpu
from jax.experimental.pallas import tpu_sc as plsc

_N = 10000
_NP = 10240
_E = 320000
_EP = 327680
_D = 128
_NG = 16
_NC = 2
_NS = 16
_NW = _NC * _NS
_EK = 128
_CPT = _EP // (_NW * _EK)
_RPT = _NP // _NS
_BN = 1024

_mesh = lambda: plsc.VectorSubcoreMesh(core_axis_name="c", subcore_axis_name="s")


def _hist_body(dst_hbm, zero_hbm, cnt_hbm, dst_v, ones_v, shared):
    cid = lax.axis_index("c")
    sid = lax.axis_index("s")
    wid = cid * _NS + sid
    pltpu.sync_copy(dst_hbm.at[pl.ds(wid * _CPT, _CPT)], dst_v)

    def _init(i, c):
        ones_v[i, :] = jnp.ones((16,), jnp.float32)
        return c
    lax.fori_loop(0, _EK, _init, 0)

    @pl.when(sid == 0)
    def _():
        pltpu.sync_copy(zero_hbm, shared)
    plsc.subcore_barrier()

    def _chunk(c, carry):
        pltpu.sync_copy(ones_v, shared.at[dst_v.at[c]], add=True)
        return carry
    lax.fori_loop(0, _CPT, _chunk, 0)
    plsc.subcore_barrier()

    @pl.when(sid == 0)
    def _():
        pltpu.sync_copy(shared, cnt_hbm.at[cid])


def _hist(dst2d):
    k = pl.kernel(
        _hist_body,
        out_type=jax.ShapeDtypeStruct((_NC, _NP, 16), jnp.float32),
        mesh=_mesh(),
        scratch_types=[
            pltpu.VMEM((_CPT, _EK), jnp.int32),
            pltpu.VMEM((_EK, 16), jnp.float32),
            pltpu.VMEM_SHARED((_NP, 16), jnp.float32),
        ],
    )
    return k(dst2d, jnp.zeros((_NP, 16), jnp.float32))


def _scat_body(y_hbm, src_hbm, dst_hbm, zero_hbm, acc_hbm, src_v, dst_v, buf0,
               acc_sh):
    cid = lax.axis_index("c")
    sid = lax.axis_index("s")
    wid = cid * _NS + sid
    pltpu.sync_copy(src_hbm.at[pl.ds(wid * _CPT, _CPT)], src_v)
    pltpu.sync_copy(dst_hbm.at[pl.ds(wid * _CPT, _CPT)], dst_v)

    @pl.when(sid == 0)
    def _():
        pltpu.sync_copy(zero_hbm, acc_sh)
    plsc.subcore_barrier()

    def _chunk(c, carry):
        pltpu.sync_copy(y_hbm.at[src_v.at[c]], buf0)
        pltpu.sync_copy(buf0, acc_sh.at[dst_v.at[c]], add=True)
        return carry
    lax.fori_loop(0, _CPT, _chunk, 0)
    plsc.subcore_barrier()

    @pl.when(sid == 0)
    def _():
        pltpu.sync_copy(acc_sh, acc_hbm.at[cid])


def _scatter(y, src2d, dst2d, zeros_nd):
    k = pl.kernel(
        _scat_body,
        out_type=jax.ShapeDtypeStruct((_NC, _NP, _D), jnp.float32),
        mesh=_mesh(),
        scratch_types=[
            pltpu.VMEM((_CPT, _EK), jnp.int32),
            pltpu.VMEM((_CPT, _EK), jnp.int32),
            pltpu.VMEM((_EK, _D), jnp.float32),
            pltpu.VMEM_SHARED((_NP, _D), jnp.float32),
        ],
    )
    return k(y, src2d, dst2d, zeros_nd)


def _dinv_block(cnt_blk):
    deg = 1.0 + cnt_blk[0, :, 0:1] + cnt_blk[1, :, 0:1]
    return lax.rsqrt(deg)


def _tc1_body(x_ref, w_ref, cnt_ref, y_ref):
    dinv = _dinv_block(cnt_ref[...])
    y_ref[...] = jnp.dot(x_ref[...], w_ref[...],
                         preferred_element_type=jnp.float32) * dinv


def _tc1(x, W1, cnt):
    return pl.pallas_call(
        _tc1_body,
        grid=(_NP // _BN,),
        in_specs=[
            pl.BlockSpec((_BN, _D), lambda i: (i, 0)),
            pl.BlockSpec((_D, _D), lambda i: (0, 0)),
            pl.BlockSpec((_NC, _BN, 16), lambda i: (0, i, 0)),
        ],
        out_specs=pl.BlockSpec((_BN, _D), lambda i: (i, 0)),
        out_shape=jax.ShapeDtypeStruct((_NP, _D), jnp.float32),
    )(x, W1, cnt)


def _tc2_body(acc_ref, y1_ref, cnt_ref, b1_ref, w2_ref, y2_ref):
    dinv = _dinv_block(cnt_ref[...])
    agg = acc_ref[0] + acc_ref[1] + y1_ref[...]
    h = jnp.maximum(agg * dinv + b1_ref[...], 0.0)
    y2_ref[...] = jnp.dot(h, w2_ref[...],
                          preferred_element_type=jnp.float32) * dinv


def _tc2(acc1, y1, cnt, b1, W2):
    return pl.pallas_call(
        _tc2_body,
        grid=(_NP // _BN,),
        in_specs=[
            pl.BlockSpec((_NC, _BN, _D), lambda i: (0, i, 0)),
            pl.BlockSpec((_BN, _D), lambda i: (i, 0)),
            pl.BlockSpec((_NC, _BN, 16), lambda i: (0, i, 0)),
            pl.BlockSpec((1, _D), lambda i: (0, 0)),
            pl.BlockSpec((_D, _D), lambda i: (0, 0)),
        ],
        out_specs=pl.BlockSpec((_BN, _D), lambda i: (i, 0)),
        out_shape=jax.ShapeDtypeStruct((_NP, _D), jnp.float32),
    )(acc1, y1, cnt, b1, W2)


def _tc3_body(acc_ref, y2_ref, cnt_ref, b2_ref, oh_ref, out_ref, cacc):
    i = pl.program_id(0)
    dinv = _dinv_block(cnt_ref[...])
    nod = (acc_ref[0] + acc_ref[1] + y2_ref[...]) * dinv + b2_ref[...]
    oh = oh_ref[...]
    dims = (((0,), (0,)), ((), ()))
    part = lax.dot_general(oh, nod, dims, preferred_element_type=jnp.float32)
    cpart = lax.dot_general(oh, jnp.ones((_BN, _D), jnp.float32), dims,
                            preferred_element_type=jnp.float32)

    @pl.when(i == 0)
    def _():
        out_ref[...] = part
        cacc[...] = cpart

    @pl.when(i > 0)
    def _():
        out_ref[...] += part
        cacc[...] += cpart

    @pl.when(i == _NP // _BN - 1)
    def _():
        out_ref[...] = out_ref[...] / jnp.maximum(cacc[...], 1.0)


def _tc3(acc2, y2, cnt, b2, oh):
    return pl.pallas_call(
        _tc3_body,
        grid=(_NP // _BN,),
        in_specs=[
            pl.BlockSpec((_NC, _BN, _D), lambda i: (0, i, 0)),
            pl.BlockSpec((_BN, _D), lambda i: (i, 0)),
            pl.BlockSpec((_NC, _BN, 16), lambda i: (0, i, 0)),
            pl.BlockSpec((1, _D), lambda i: (0, 0)),
            pl.BlockSpec((_BN, _NG), lambda i: (i, 0)),
        ],
        out_specs=pl.BlockSpec((_NG, _D), lambda i: (0, 0)),
        out_shape=jax.ShapeDtypeStruct((_NG, _D), jnp.float32),
        scratch_shapes=[pltpu.VMEM((_NG, _D), jnp.float32)],
    )(acc2, y2, cnt, b2, oh)


def kernel(x, edge_index, batch, W1, b1, W2, b2):
    ei = edge_index.astype(jnp.int32)
    pad_e = jnp.full((_EP - _E,), _N, jnp.int32)
    src2d = jnp.concatenate([ei[0], pad_e]).reshape(_NW * _CPT, _EK)
    dst2d = jnp.concatenate([ei[1], pad_e]).reshape(_NW * _CPT, _EK)
    x_p = jnp.pad(x, ((0, _NP - _N), (0, 0)))
    batch_p = jnp.pad(batch.astype(jnp.int32), (0, _NP - _N),
                      constant_values=_NG)
    oh = (batch_p[:, None] == jnp.arange(_NG, dtype=jnp.int32)[None, :])
    oh = oh.astype(jnp.float32)

    zeros_nd = jnp.zeros((_NP, _D), jnp.float32)
    cnt = _hist(dst2d)
    y1 = _tc1(x_p, W1, cnt)
    acc1 = _scatter(y1, src2d, dst2d, zeros_nd)
    y2 = _tc2(acc1, y1, cnt, b1.reshape(1, _D), W2)
    acc2 = _scatter(y2, src2d, dst2d, zeros_nd)
    return _tc3(acc2, y2, cnt, b2.reshape(1, _D), oh)


# speedup vs baseline: 10.3417x; 1.0840x over previous
import jax
import jax.numpy as jnp
from jax import lax
from jax.experimental import pallas as pl
from jax.experimental.pallas import tpu as pltpu
from jax.experimental.pallas import tpu_sc as plsc

_N = 10000
_NP = 10240
_E = 320000
_EP = 327680
_D = 128
_NG = 16
_NC = 2
_NS = 16
_NW = _NC * _NS
_EK = 128
_CPT = _EP // (_NW * _EK)
_RPT = _NP // _NS
_BN = 1024

_mesh = lambda: plsc.VectorSubcoreMesh(core_axis_name="c", subcore_axis_name="s")


def _hist_body(dst_hbm, zero_hbm, cnt_hbm, dst_v, ones_v, shared):
    cid = lax.axis_index("c")
    sid = lax.axis_index("s")
    wid = cid * _NS + sid
    pltpu.sync_copy(dst_hbm.at[pl.ds(wid * _CPT, _CPT)], dst_v)

    def _init(i, c):
        ones_v[i, :] = jnp.ones((16,), jnp.float32)
        return c
    lax.fori_loop(0, _EK, _init, 0)

    @pl.when(sid == 0)
    def _():
        pltpu.sync_copy(zero_hbm, shared)
    plsc.subcore_barrier()

    def _chunk(c, carry):
        pltpu.sync_copy(ones_v, shared.at[dst_v.at[c]], add=True)
        return carry
    lax.fori_loop(0, _CPT, _chunk, 0)
    plsc.subcore_barrier()

    @pl.when(sid == 0)
    def _():
        pltpu.sync_copy(shared, cnt_hbm.at[cid])


def _hist(dst2d):
    k = pl.kernel(
        _hist_body,
        out_type=jax.ShapeDtypeStruct((_NC, _NP, 16), jnp.float32),
        mesh=_mesh(),
        scratch_types=[
            pltpu.VMEM((_CPT, _EK), jnp.int32),
            pltpu.VMEM((_EK, 16), jnp.float32),
            pltpu.VMEM_SHARED((_NP, 16), jnp.float32),
        ],
    )
    return k(dst2d, jnp.zeros((_NP, 16), jnp.float32))


def _scat_body(y_hbm, src_hbm, dst_hbm, zero_hbm, acc_hbm, src_v, dst_v, buf0,
               buf1, acc_sh, sem0, sem1):
    cid = lax.axis_index("c")
    sid = lax.axis_index("s")
    wid = cid * _NS + sid

    @pl.when(sid == 0)
    def _():
        pltpu.sync_copy(zero_hbm, acc_sh)
    plsc.subcore_barrier()

    for h in range(2):
        base = wid * _CPT + h * (_CPT // 2)
        pltpu.sync_copy(src_hbm.at[pl.ds(base, _CPT // 2)], src_v)
        pltpu.sync_copy(dst_hbm.at[pl.ds(base, _CPT // 2)], dst_v)
        pltpu.async_copy(y_hbm.at[src_v.at[0]], buf0, sem0)

        def _pair(t, carry):
            c0 = 2 * t
            pltpu.make_async_copy(y_hbm.at[src_v.at[c0]], buf0, sem0).wait()
            d1 = pltpu.async_copy(y_hbm.at[src_v.at[c0 + 1]], buf1, sem1)
            pltpu.sync_copy(buf0, acc_sh.at[dst_v.at[c0]], add=True)
            d1.wait()

            @pl.when(t < _CPT // 4 - 1)
            def _():
                pltpu.async_copy(y_hbm.at[src_v.at[c0 + 2]], buf0, sem0)
            pltpu.sync_copy(buf1, acc_sh.at[dst_v.at[c0 + 1]], add=True)
            return carry
        lax.fori_loop(0, _CPT // 4, _pair, 0)
    plsc.subcore_barrier()

    @pl.when(sid == 0)
    def _():
        pltpu.sync_copy(acc_sh, acc_hbm.at[cid])


def _scatter(y, src2d, dst2d, zeros_nd):
    k = pl.kernel(
        _scat_body,
        out_type=jax.ShapeDtypeStruct((_NC, _NP, _D), jnp.float32),
        mesh=_mesh(),
        scratch_types=[
            pltpu.VMEM((_CPT // 2, _EK), jnp.int32),
            pltpu.VMEM((_CPT // 2, _EK), jnp.int32),
            pltpu.VMEM((_EK, _D), jnp.float32),
            pltpu.VMEM((_EK, _D), jnp.float32),
            pltpu.VMEM_SHARED((_NP, _D), jnp.float32),
            pltpu.SemaphoreType.DMA,
            pltpu.SemaphoreType.DMA,
        ],
    )
    return k(y, src2d, dst2d, zeros_nd)


def _dinv_block(cnt_blk):
    deg = 1.0 + cnt_blk[0, :, 0:1] + cnt_blk[1, :, 0:1]
    return lax.rsqrt(deg)


def _tc1_body(x_ref, w_ref, cnt_ref, y_ref):
    dinv = _dinv_block(cnt_ref[...])
    y_ref[...] = jnp.dot(x_ref[...], w_ref[...],
                         preferred_element_type=jnp.float32) * dinv


def _tc1(x, W1, cnt):
    return pl.pallas_call(
        _tc1_body,
        grid=(_NP // _BN,),
        in_specs=[
            pl.BlockSpec((_BN, _D), lambda i: (i, 0)),
            pl.BlockSpec((_D, _D), lambda i: (0, 0)),
            pl.BlockSpec((_NC, _BN, 16), lambda i: (0, i, 0)),
        ],
        out_specs=pl.BlockSpec((_BN, _D), lambda i: (i, 0)),
        out_shape=jax.ShapeDtypeStruct((_NP, _D), jnp.float32),
    )(x, W1, cnt)


def _tc2_body(acc_ref, y1_ref, cnt_ref, b1_ref, w2_ref, y2_ref):
    dinv = _dinv_block(cnt_ref[...])
    agg = acc_ref[0] + acc_ref[1] + y1_ref[...]
    h = jnp.maximum(agg * dinv + b1_ref[...], 0.0)
    y2_ref[...] = jnp.dot(h, w2_ref[...],
                          preferred_element_type=jnp.float32) * dinv


def _tc2(acc1, y1, cnt, b1, W2):
    return pl.pallas_call(
        _tc2_body,
        grid=(_NP // _BN,),
        in_specs=[
            pl.BlockSpec((_NC, _BN, _D), lambda i: (0, i, 0)),
            pl.BlockSpec((_BN, _D), lambda i: (i, 0)),
            pl.BlockSpec((_NC, _BN, 16), lambda i: (0, i, 0)),
            pl.BlockSpec((1, _D), lambda i: (0, 0)),
            pl.BlockSpec((_D, _D), lambda i: (0, 0)),
        ],
        out_specs=pl.BlockSpec((_BN, _D), lambda i: (i, 0)),
        out_shape=jax.ShapeDtypeStruct((_NP, _D), jnp.float32),
    )(acc1, y1, cnt, b1, W2)


def _tc3_body(acc_ref, y2_ref, cnt_ref, b2_ref, oh_ref, out_ref, cacc):
    i = pl.program_id(0)
    dinv = _dinv_block(cnt_ref[...])
    nod = (acc_ref[0] + acc_ref[1] + y2_ref[...]) * dinv + b2_ref[...]
    oh = oh_ref[...]
    dims = (((0,), (0,)), ((), ()))
    part = lax.dot_general(oh, nod, dims, preferred_element_type=jnp.float32)
    cpart = lax.dot_general(oh, jnp.ones((_BN, _D), jnp.float32), dims,
                            preferred_element_type=jnp.float32)

    @pl.when(i == 0)
    def _():
        out_ref[...] = part
        cacc[...] = cpart

    @pl.when(i > 0)
    def _():
        out_ref[...] += part
        cacc[...] += cpart

    @pl.when(i == _NP // _BN - 1)
    def _():
        out_ref[...] = out_ref[...] / jnp.maximum(cacc[...], 1.0)


def _tc3(acc2, y2, cnt, b2, oh):
    return pl.pallas_call(
        _tc3_body,
        grid=(_NP // _BN,),
        in_specs=[
            pl.BlockSpec((_NC, _BN, _D), lambda i: (0, i, 0)),
            pl.BlockSpec((_BN, _D), lambda i: (i, 0)),
            pl.BlockSpec((_NC, _BN, 16), lambda i: (0, i, 0)),
            pl.BlockSpec((1, _D), lambda i: (0, 0)),
            pl.BlockSpec((_BN, _NG), lambda i: (i, 0)),
        ],
        out_specs=pl.BlockSpec((_NG, _D), lambda i: (0, 0)),
        out_shape=jax.ShapeDtypeStruct((_NG, _D), jnp.float32),
        scratch_shapes=[pltpu.VMEM((_NG, _D), jnp.float32)],
    )(acc2, y2, cnt, b2, oh)


def kernel(x, edge_index, batch, W1, b1, W2, b2):
    ei = edge_index.astype(jnp.int32)
    pad_e = jnp.full((_EP - _E,), _N, jnp.int32)
    src2d = jnp.concatenate([ei[0], pad_e]).reshape(_NW * _CPT, _EK)
    dst2d = jnp.concatenate([ei[1], pad_e]).reshape(_NW * _CPT, _EK)
    x_p = jnp.pad(x, ((0, _NP - _N), (0, 0)))
    batch_p = jnp.pad(batch.astype(jnp.int32), (0, _NP - _N),
                      constant_values=_NG)
    oh = (batch_p[:, None] == jnp.arange(_NG, dtype=jnp.int32)[None, :])
    oh = oh.astype(jnp.float32)

    zeros_nd = jnp.zeros((_NP, _D), jnp.float32)
    cnt = _hist(dst2d)
    y1 = _tc1(x_p, W1, cnt)
    acc1 = _scatter(y1, src2d, dst2d, zeros_nd)
    y2 = _tc2(acc1, y1, cnt, b1.reshape(1, _D), W2)
    acc2 = _scatter(y2, src2d, dst2d, zeros_nd)
    return _tc3(acc2, y2, cnt, b2.reshape(1, _D), oh)


# speedup vs baseline: 27.8693x; 2.6948x over previous
import jax
import jax.numpy as jnp
from jax import lax
from jax.experimental import pallas as pl
from jax.experimental.pallas import tpu as pltpu
from jax.experimental.pallas import tpu_sc as plsc

_N = 10000
_NP = 10240
_E = 320000
_EP = 327680
_D = 128
_NG = 16
_NC = 2
_NS = 16
_NW = _NC * _NS
_EK = 128
_CPT = _EP // (_NW * _EK)
_RPT = _NP // _NS
_BN = 1024

_mesh = lambda: plsc.VectorSubcoreMesh(core_axis_name="c", subcore_axis_name="s")


def _hist_body(dst_hbm, zero_hbm, cnt_hbm, dst_v, ones_v, shared):
    cid = lax.axis_index("c")
    sid = lax.axis_index("s")
    wid = cid * _NS + sid
    pltpu.sync_copy(dst_hbm.at[pl.ds(wid * _CPT, _CPT)], dst_v)

    def _init(i, c):
        ones_v[i, :] = jnp.ones((16,), jnp.float32)
        return c
    lax.fori_loop(0, _EK, _init, 0)

    @pl.when(sid == 0)
    def _():
        pltpu.sync_copy(zero_hbm, shared)
    plsc.subcore_barrier()

    def _chunk(c, carry):
        pltpu.sync_copy(ones_v, shared.at[dst_v.at[c]], add=True)
        return carry
    lax.fori_loop(0, _CPT, _chunk, 0)
    plsc.subcore_barrier()

    @pl.when(sid == 0)
    def _():
        pltpu.sync_copy(shared, cnt_hbm.at[cid])


def _hist(dst2d):
    k = pl.kernel(
        _hist_body,
        out_type=jax.ShapeDtypeStruct((_NC, _NP, 16), jnp.float32),
        mesh=_mesh(),
        scratch_types=[
            pltpu.VMEM((_CPT, _EK), jnp.int32),
            pltpu.VMEM((_EK, 16), jnp.float32),
            pltpu.VMEM_SHARED((_NP, 16), jnp.float32),
        ],
    )
    return k(dst2d, jnp.zeros((_NP, 16), jnp.float32))


def _scat_body(y_hbm, src_hbm, dst_hbm, zero_hbm, acc_hbm, src_v, dst_v, buf0,
               buf1, acc_sh, sem0, sem1):
    cid = lax.axis_index("c")
    sid = lax.axis_index("s")
    wid = cid * _NS + sid

    @pl.when(sid == 0)
    def _():
        pltpu.sync_copy(zero_hbm, acc_sh)
    plsc.subcore_barrier()

    for h in range(2):
        base = wid * _CPT + h * (_CPT // 2)
        pltpu.sync_copy(src_hbm.at[pl.ds(base, _CPT // 2)], src_v)
        pltpu.sync_copy(dst_hbm.at[pl.ds(base, _CPT // 2)], dst_v)
        pltpu.async_copy(y_hbm.at[src_v.at[0]], buf0, sem0)

        def _pair(t, carry):
            c0 = 2 * t
            pltpu.make_async_copy(y_hbm.at[src_v.at[c0]], buf0, sem0).wait()
            d1 = pltpu.async_copy(y_hbm.at[src_v.at[c0 + 1]], buf1, sem1)
            pltpu.sync_copy(buf0, acc_sh.at[dst_v.at[c0]], add=True)
            d1.wait()

            @pl.when(t < _CPT // 4 - 1)
            def _():
                pltpu.async_copy(y_hbm.at[src_v.at[c0 + 2]], buf0, sem0)
            pltpu.sync_copy(buf1, acc_sh.at[dst_v.at[c0 + 1]], add=True)
            return carry
        lax.fori_loop(0, _CPT // 4, _pair, 0)
    plsc.subcore_barrier()

    @pl.when(sid == 0)
    def _():
        pltpu.sync_copy(acc_sh, acc_hbm.at[cid])


def _scatter(y, src2d, dst2d, zeros_nd):
    k = pl.kernel(
        _scat_body,
        out_type=jax.ShapeDtypeStruct((_NC, _NP, _D), jnp.float32),
        mesh=_mesh(),
        scratch_types=[
            pltpu.VMEM((_CPT // 2, _EK), jnp.int32),
            pltpu.VMEM((_CPT // 2, _EK), jnp.int32),
            pltpu.VMEM((_EK, _D), jnp.float32),
            pltpu.VMEM((_EK, _D), jnp.float32),
            pltpu.VMEM_SHARED((_NP, _D), jnp.float32),
            pltpu.SemaphoreType.DMA,
            pltpu.SemaphoreType.DMA,
        ],
    )
    return k(y, src2d, dst2d, zeros_nd)


def _dinv_block(cnt_blk):
    deg = 1.0 + cnt_blk[0, :, 0:1] + cnt_blk[1, :, 0:1]
    return lax.rsqrt(deg)


def _tc1_body(x_ref, w_ref, cnt_ref, y_ref):
    dinv = _dinv_block(cnt_ref[...])
    y_ref[...] = jnp.dot(x_ref[...], w_ref[...],
                         preferred_element_type=jnp.float32) * dinv


def _tc1(x, W1, cnt):
    return pl.pallas_call(
        _tc1_body,
        grid=(_NP // _BN,),
        in_specs=[
            pl.BlockSpec((_BN, _D), lambda i: (i, 0)),
            pl.BlockSpec((_D, _D), lambda i: (0, 0)),
            pl.BlockSpec((_NC, _BN, 16), lambda i: (0, i, 0)),
        ],
        out_specs=pl.BlockSpec((_BN, _D), lambda i: (i, 0)),
        out_shape=jax.ShapeDtypeStruct((_NP, _D), jnp.float32),
    )(x, W1, cnt)


def _tc2_body(acc_ref, y1_ref, cnt_ref, b1_ref, w2_ref, y2_ref):
    dinv = _dinv_block(cnt_ref[...])
    agg = acc_ref[0] + acc_ref[1] + y1_ref[...]
    h = jnp.maximum(agg * dinv + b1_ref[...], 0.0)
    y2_ref[...] = jnp.dot(h, w2_ref[...],
                          preferred_element_type=jnp.float32) * dinv


def _tc2(acc1, y1, cnt, b1, W2):
    return pl.pallas_call(
        _tc2_body,
        grid=(_NP // _BN,),
        in_specs=[
            pl.BlockSpec((_NC, _BN, _D), lambda i: (0, i, 0)),
            pl.BlockSpec((_BN, _D), lambda i: (i, 0)),
            pl.BlockSpec((_NC, _BN, 16), lambda i: (0, i, 0)),
            pl.BlockSpec((1, _D), lambda i: (0, 0)),
            pl.BlockSpec((_D, _D), lambda i: (0, 0)),
        ],
        out_specs=pl.BlockSpec((_BN, _D), lambda i: (i, 0)),
        out_shape=jax.ShapeDtypeStruct((_NP, _D), jnp.float32),
    )(acc1, y1, cnt, b1, W2)


def _tc3_body(acc_ref, y2_ref, cnt_ref, b2_ref, oh_ref, out_ref, cacc):
    i = pl.program_id(0)
    dinv = _dinv_block(cnt_ref[...])
    nod = (acc_ref[0] + acc_ref[1] + y2_ref[...]) * dinv + b2_ref[...]
    oh = oh_ref[...]
    dims = (((0,), (0,)), ((), ()))
    part = lax.dot_general(oh, nod, dims, preferred_element_type=jnp.float32)
    cpart = lax.dot_general(oh, jnp.ones((_BN, _D), jnp.float32), dims,
                            preferred_element_type=jnp.float32)

    @pl.when(i == 0)
    def _():
        out_ref[...] = part
        cacc[...] = cpart

    @pl.when(i > 0)
    def _():
        out_ref[...] += part
        cacc[...] += cpart

    @pl.when(i == _NP // _BN - 1)
    def _():
        out_ref[...] = out_ref[...] / jnp.maximum(cacc[...], 1.0)


def _tc3(acc2, y2, cnt, b2, oh):
    return pl.pallas_call(
        _tc3_body,
        grid=(_NP // _BN,),
        in_specs=[
            pl.BlockSpec((_NC, _BN, _D), lambda i: (0, i, 0)),
            pl.BlockSpec((_BN, _D), lambda i: (i, 0)),
            pl.BlockSpec((_NC, _BN, 16), lambda i: (0, i, 0)),
            pl.BlockSpec((1, _D), lambda i: (0, 0)),
            pl.BlockSpec((_BN, _NG), lambda i: (i, 0)),
        ],
        out_specs=pl.BlockSpec((_NG, _D), lambda i: (0, 0)),
        out_shape=jax.ShapeDtypeStruct((_NG, _D), jnp.float32),
        scratch_shapes=[pltpu.VMEM((_NG, _D), jnp.float32)],
    )(acc2, y2, cnt, b2, oh)


def kernel(x, edge_index, batch, W1, b1, W2, b2):
    ei = edge_index.astype(jnp.int32)
    pad_e = _N + jnp.arange(_EP - _E, dtype=jnp.int32) % (_NP - _N)
    src2d = jnp.concatenate([ei[0], pad_e]).reshape(_NW * _CPT, _EK)
    dst2d = jnp.concatenate([ei[1], pad_e]).reshape(_NW * _CPT, _EK)
    x_p = jnp.pad(x, ((0, _NP - _N), (0, 0)))
    batch_p = jnp.pad(batch.astype(jnp.int32), (0, _NP - _N),
                      constant_values=_NG)
    oh = (batch_p[:, None] == jnp.arange(_NG, dtype=jnp.int32)[None, :])
    oh = oh.astype(jnp.float32)

    zeros_nd = jnp.zeros((_NP, _D), jnp.float32)
    cnt = _hist(dst2d)
    y1 = _tc1(x_p, W1, cnt)
    acc1 = _scatter(y1, src2d, dst2d, zeros_nd)
    y2 = _tc2(acc1, y1, cnt, b1.reshape(1, _D), W2)
    acc2 = _scatter(y2, src2d, dst2d, zeros_nd)
    return _tc3(acc2, y2, cnt, b2.reshape(1, _D), oh)


# speedup vs baseline: 27.8769x; 1.0003x over previous
import jax
import jax.numpy as jnp
from jax import lax
from jax.experimental import pallas as pl
from jax.experimental.pallas import tpu as pltpu
from jax.experimental.pallas import tpu_sc as plsc

_N = 10000
_NP = 10240
_E = 320000
_EP = 327680
_D = 128
_NG = 16
_NC = 2
_NS = 16
_NW = _NC * _NS
_EK = 128
_CPT = _EP // (_NW * _EK)
_RPT = _NP // _NS
_BN = 1024

_mesh = lambda: plsc.VectorSubcoreMesh(core_axis_name="c", subcore_axis_name="s")


def _hist_body(dst_hbm, zero_hbm, cnt_hbm, dst_v, ones_v, shared):
    cid = lax.axis_index("c")
    sid = lax.axis_index("s")
    wid = cid * _NS + sid
    pltpu.sync_copy(dst_hbm.at[pl.ds(wid * _CPT, _CPT)], dst_v)

    def _init(i, c):
        ones_v[i, :] = jnp.ones((16,), jnp.float32)
        return c
    lax.fori_loop(0, _EK, _init, 0)

    @pl.when(sid == 0)
    def _():
        pltpu.sync_copy(zero_hbm, shared)
    plsc.subcore_barrier()

    def _chunk(c, carry):
        pltpu.sync_copy(ones_v, shared.at[dst_v.at[c]], add=True)
        return carry
    lax.fori_loop(0, _CPT, _chunk, 0)
    plsc.subcore_barrier()

    @pl.when(sid == 0)
    def _():
        pltpu.sync_copy(shared, cnt_hbm.at[cid])


def _hist(dst2d):
    k = pl.kernel(
        _hist_body,
        out_type=jax.ShapeDtypeStruct((_NC, _NP, 16), jnp.float32),
        mesh=_mesh(),
        scratch_types=[
            pltpu.VMEM((_CPT, _EK), jnp.int32),
            pltpu.VMEM((_EK, 16), jnp.float32),
            pltpu.VMEM_SHARED((_NP, 16), jnp.float32),
        ],
    )
    return k(dst2d, jnp.zeros((_NP, 16), jnp.float32))


def _scat_body(y_hbm, src_hbm, dst_hbm, zero_hbm, acc_hbm, src_v, dst_v, buf0,
               buf1, acc_sh, sem0, sem1, sems0, sems1):
    cid = lax.axis_index("c")
    sid = lax.axis_index("s")
    wid = cid * _NS + sid

    @pl.when(sid == 0)
    def _():
        pltpu.sync_copy(zero_hbm, acc_sh)
    plsc.subcore_barrier()

    for h in range(2):
        base = wid * _CPT + h * (_CPT // 2)
        pltpu.sync_copy(src_hbm.at[pl.ds(base, _CPT // 2)], src_v)
        pltpu.sync_copy(dst_hbm.at[pl.ds(base, _CPT // 2)], dst_v)
        pltpu.async_copy(y_hbm.at[src_v.at[0]], buf0, sem0)

        def _pair(t, carry):
            c0 = 2 * t
            pltpu.make_async_copy(y_hbm.at[src_v.at[c0]], buf0, sem0).wait()
            pltpu.async_copy(buf0, acc_sh.at[dst_v.at[c0]], sems0, add=True)

            @pl.when(t > 0)
            def _():
                pltpu.make_async_copy(buf1, acc_sh.at[dst_v.at[c0]],
                                      sems1).wait()
            pltpu.async_copy(y_hbm.at[src_v.at[c0 + 1]], buf1, sem1)
            pltpu.make_async_copy(y_hbm.at[src_v.at[c0 + 1]], buf1, sem1).wait()
            pltpu.async_copy(buf1, acc_sh.at[dst_v.at[c0 + 1]], sems1, add=True)

            @pl.when(t < _CPT // 4 - 1)
            def _():
                pltpu.make_async_copy(buf0, acc_sh.at[dst_v.at[c0]],
                                      sems0).wait()
                pltpu.async_copy(y_hbm.at[src_v.at[c0 + 2]], buf0, sem0)
            return carry
        lax.fori_loop(0, _CPT // 4, _pair, 0)
        pltpu.make_async_copy(buf0, acc_sh.at[dst_v.at[0]], sems0).wait()
        pltpu.make_async_copy(buf1, acc_sh.at[dst_v.at[0]], sems1).wait()
    plsc.subcore_barrier()

    @pl.when(sid == 0)
    def _():
        pltpu.sync_copy(acc_sh, acc_hbm.at[cid])


def _scatter(y, src2d, dst2d, zeros_nd):
    k = pl.kernel(
        _scat_body,
        out_type=jax.ShapeDtypeStruct((_NC, _NP, _D), jnp.float32),
        mesh=_mesh(),
        scratch_types=[
            pltpu.VMEM((_CPT // 2, _EK), jnp.int32),
            pltpu.VMEM((_CPT // 2, _EK), jnp.int32),
            pltpu.VMEM((_EK, _D), jnp.float32),
            pltpu.VMEM((_EK, _D), jnp.float32),
            pltpu.VMEM_SHARED((_NP, _D), jnp.float32),
            pltpu.SemaphoreType.DMA,
            pltpu.SemaphoreType.DMA,
            pltpu.SemaphoreType.DMA,
            pltpu.SemaphoreType.DMA,
        ],
    )
    return k(y, src2d, dst2d, zeros_nd)


def _dinv_block(cnt_blk):
    deg = 1.0 + cnt_blk[0, :, 0:1] + cnt_blk[1, :, 0:1]
    return lax.rsqrt(deg)


def _tc1_body(x_ref, w_ref, cnt_ref, y_ref):
    dinv = _dinv_block(cnt_ref[...])
    y_ref[...] = jnp.dot(x_ref[...], w_ref[...],
                         preferred_element_type=jnp.float32) * dinv


def _tc1(x, W1, cnt):
    return pl.pallas_call(
        _tc1_body,
        grid=(_NP // _BN,),
        in_specs=[
            pl.BlockSpec((_BN, _D), lambda i: (i, 0)),
            pl.BlockSpec((_D, _D), lambda i: (0, 0)),
            pl.BlockSpec((_NC, _BN, 16), lambda i: (0, i, 0)),
        ],
        out_specs=pl.BlockSpec((_BN, _D), lambda i: (i, 0)),
        out_shape=jax.ShapeDtypeStruct((_NP, _D), jnp.float32),
    )(x, W1, cnt)


def _tc2_body(acc_ref, y1_ref, cnt_ref, b1_ref, w2_ref, y2_ref):
    dinv = _dinv_block(cnt_ref[...])
    agg = acc_ref[0] + acc_ref[1] + y1_ref[...]
    h = jnp.maximum(agg * dinv + b1_ref[...], 0.0)
    y2_ref[...] = jnp.dot(h, w2_ref[...],
                          preferred_element_type=jnp.float32) * dinv


def _tc2(acc1, y1, cnt, b1, W2):
    return pl.pallas_call(
        _tc2_body,
        grid=(_NP // _BN,),
        in_specs=[
            pl.BlockSpec((_NC, _BN, _D), lambda i: (0, i, 0)),
            pl.BlockSpec((_BN, _D), lambda i: (i, 0)),
            pl.BlockSpec((_NC, _BN, 16), lambda i: (0, i, 0)),
            pl.BlockSpec((1, _D), lambda i: (0, 0)),
            pl.BlockSpec((_D, _D), lambda i: (0, 0)),
        ],
        out_specs=pl.BlockSpec((_BN, _D), lambda i: (i, 0)),
        out_shape=jax.ShapeDtypeStruct((_NP, _D), jnp.float32),
    )(acc1, y1, cnt, b1, W2)


def _tc3_body(acc_ref, y2_ref, cnt_ref, b2_ref, oh_ref, out_ref, cacc):
    i = pl.program_id(0)
    dinv = _dinv_block(cnt_ref[...])
    nod = (acc_ref[0] + acc_ref[1] + y2_ref[...]) * dinv + b2_ref[...]
    oh = oh_ref[...]
    dims = (((0,), (0,)), ((), ()))
    part = lax.dot_general(oh, nod, dims, preferred_element_type=jnp.float32)
    cpart = lax.dot_general(oh, jnp.ones((_BN, _D), jnp.float32), dims,
                            preferred_element_type=jnp.float32)

    @pl.when(i == 0)
    def _():
        out_ref[...] = part
        cacc[...] = cpart

    @pl.when(i > 0)
    def _():
        out_ref[...] += part
        cacc[...] += cpart

    @pl.when(i == _NP // _BN - 1)
    def _():
        out_ref[...] = out_ref[...] / jnp.maximum(cacc[...], 1.0)


def _tc3(acc2, y2, cnt, b2, oh):
    return pl.pallas_call(
        _tc3_body,
        grid=(_NP // _BN,),
        in_specs=[
            pl.BlockSpec((_NC, _BN, _D), lambda i: (0, i, 0)),
            pl.BlockSpec((_BN, _D), lambda i: (i, 0)),
            pl.BlockSpec((_NC, _BN, 16), lambda i: (0, i, 0)),
            pl.BlockSpec((1, _D), lambda i: (0, 0)),
            pl.BlockSpec((_BN, _NG), lambda i: (i, 0)),
        ],
        out_specs=pl.BlockSpec((_NG, _D), lambda i: (0, 0)),
        out_shape=jax.ShapeDtypeStruct((_NG, _D), jnp.float32),
        scratch_shapes=[pltpu.VMEM((_NG, _D), jnp.float32)],
    )(acc2, y2, cnt, b2, oh)


def kernel(x, edge_index, batch, W1, b1, W2, b2):
    ei = edge_index.astype(jnp.int32)
    pad_e = _N + jnp.arange(_EP - _E, dtype=jnp.int32) % (_NP - _N)
    src2d = jnp.concatenate([ei[0], pad_e]).reshape(_NW * _CPT, _EK)
    dst2d = jnp.concatenate([ei[1], pad_e]).reshape(_NW * _CPT, _EK)
    x_p = jnp.pad(x, ((0, _NP - _N), (0, 0)))
    batch_p = jnp.pad(batch.astype(jnp.int32), (0, _NP - _N),
                      constant_values=_NG)
    oh = (batch_p[:, None] == jnp.arange(_NG, dtype=jnp.int32)[None, :])
    oh = oh.astype(jnp.float32)

    zeros_nd = jnp.zeros((_NP, _D), jnp.float32)
    cnt = _hist(dst2d)
    y1 = _tc1(x_p, W1, cnt)
    acc1 = _scatter(y1, src2d, dst2d, zeros_nd)
    y2 = _tc2(acc1, y1, cnt, b1.reshape(1, _D), W2)
    acc2 = _scatter(y2, src2d, dst2d, zeros_nd)
    return _tc3(acc2, y2, cnt, b2.reshape(1, _D), oh)
